# Initial kernel scaffold; baseline (speedup 1.0000x reference)
import jax
import jax.numpy as jnp
from jax.experimental import pallas as pl


def kernel(x, edge_index, edge_attr, W_node, b_node, W_edge, b_edge, W1, b1, W2, b2, We1, be1, We2, be2, Wc1, bc1, Wc2, bc2):
    raise NotImplementedError("write your pallas kernel here")


# speedup vs baseline: 1.1354x; 1.1354x over previous
import jax
import jax.numpy as jnp
from jax.experimental import pallas as pl


def _head_kernel(h_ref, wc1_ref, bc1_ref, wc2_ref, bc2_ref, o_ref):
    z = jnp.maximum(jnp.dot(h_ref[...], wc1_ref[...],
                            preferred_element_type=jnp.float32) + bc1_ref[...], 0.0)
    o_ref[...] = jnp.dot(z, wc2_ref[...], preferred_element_type=jnp.float32) + bc2_ref[...]


def kernel(x, edge_index, edge_attr, W_node, b_node, W_edge, b_edge, W1, b1, W2, b2, We1, be1, We2, be2, Wc1, bc1, Wc2, bc2):
    src = edge_index[0]
    dst = edge_index[1]
    h = x @ W_node + b_node
    e = edge_attr @ W_edge + b_edge
    L = W1.shape[0]
    for i in range(L):
        msg = jax.nn.relu(h[src] + e)
        agg = jnp.zeros_like(h).at[dst].add(msg)
        z = h + agg
        z = jax.nn.relu(z @ W1[i] + b1[i]) @ W2[i] + b2[i]
        mu = jnp.mean(z, axis=0)
        var = jnp.var(z, axis=0)
        z = (z - mu) / jnp.sqrt(var + 1e-5)
        h = (h + jax.nn.relu(z)) / 2.0
        if i < L - 1:
            eu = jnp.concatenate([h[src], h[dst], e], axis=-1)
            eu = jax.nn.relu(eu @ We1[i] + be1[i]) @ We2[i] + be2[i]
            e = (e + eu) / 2.0
    NCLS = Wc2.shape[1]
    Wc2p = jnp.pad(Wc2, ((0, 0), (0, 128 - NCLS)))
    bc2p = jnp.pad(bc2, (0, 128 - NCLS))
    out = pl.pallas_call(
        _head_kernel,
        out_shape=jax.ShapeDtypeStruct((h.shape[0], 128), jnp.float32),
    )(h, Wc1, bc1[None, :], Wc2p, bc2p[None, :])
    return out[:, :NCLS]


# speedup vs baseline: 2.3467x; 2.0669x over previous
import functools

import jax
import jax.numpy as jnp
from jax import lax
from jax.experimental import pallas as pl
from jax.experimental.pallas import tpu as pltpu
from jax.experimental.pallas import tpu_sc as plsc

_N = 10000
_E = 160000
_H = 256
_EPAD = 163840
_KCH = 128
_NW = 32
_PER_W = _EPAD // _NW
_CH_W = _PER_W // _KCH
_PER_S = _EPAD // 16
_CH_S = _PER_S // _KCH
_NPAD = 10240
_TN = 1000
_NT = _N // _TN
_TE = 2048
_TEM = 1024
_F32 = jnp.float32


def _sc_mesh():
    return plsc.VectorSubcoreMesh(core_axis_name="c", subcore_axis_name="s")


def _sc_gather_x(xf, srcp):
    import dataclasses
    cp = pltpu.CompilerParams()
    if "needs_layout_passes" in pltpu.CompilerParams.__dataclass_fields__:
        cp = dataclasses.replace(cp, needs_layout_passes=False)

    @functools.partial(
        pl.kernel, mesh=_sc_mesh(),
        out_type=jax.ShapeDtypeStruct((_EPAD,), _F32),
        compiler_params=cp,
        scratch_types=[pltpu.VMEM((_N,), _F32),
                       pltpu.VMEM((_KCH,), jnp.int32),
                       pltpu.VMEM((_KCH,), _F32),
                       pltpu.SemaphoreType.DMA],
    )
    def k(x_hbm, s_hbm, o_hbm, xv, iv, ov, sem):
        w = lax.axis_index("s") * 2 + lax.axis_index("c")
        base = w * _PER_W
        pltpu.sync_copy(x_hbm, xv)

        @pl.loop(0, _CH_W)
        def _(ch):
            off = base + ch * _KCH
            pltpu.sync_copy(s_hbm.at[pl.ds(off, _KCH)], iv)

            @pl.loop(0, _KCH // 16)
            def _(j):
                idx = iv[pl.ds(j * 16, 16)]
                ov[pl.ds(j * 16, 16)] = plsc.load_gather(xv, [idx])

            pltpu.sync_copy(ov, o_hbm.at[pl.ds(off, _KCH)])

    return k(xf, srcp)


def _sc_gather_tq(T, Q, srcp, dstp):
    @functools.partial(
        pl.kernel, mesh=_sc_mesh(),
        out_type=[jax.ShapeDtypeStruct((_EPAD, 512), _F32),
                  jax.ShapeDtypeStruct((_EPAD, 256), _F32)],
        scratch_types=[pltpu.VMEM((_KCH,), jnp.int32),
                       pltpu.VMEM((_KCH,), jnp.int32),
                       pltpu.VMEM((_KCH, 512), _F32),
                       pltpu.VMEM((_KCH, 256), _F32),
                       pltpu.SemaphoreType.DMA,
                       pltpu.SemaphoreType.DMA],
    )
    def k(t_hbm, q_hbm, s_hbm, d_hbm, tg_hbm, qd_hbm, iv1, iv2, rt, rq, sem1, sem2):
        w = lax.axis_index("s") * 2 + lax.axis_index("c")
        base = w * _PER_W

        @pl.loop(0, _CH_W)
        def _(ch):
            off = base + ch * _KCH
            pltpu.sync_copy(s_hbm.at[pl.ds(off, _KCH)], iv1)
            pltpu.sync_copy(d_hbm.at[pl.ds(off, _KCH)], iv2)
            cp1 = pltpu.async_copy(t_hbm.at[iv1], rt, sem1)
            cp2 = pltpu.async_copy(q_hbm.at[iv2], rq, sem2)
            cp1.wait()
            pltpu.sync_copy(rt, tg_hbm.at[pl.ds(off, _KCH)])
            cp2.wait()
            pltpu.sync_copy(rq, qd_hbm.at[pl.ds(off, _KCH)])

    return k(T, Q, srcp, dstp)


def _sc_scatter_add(msg2, dstp, zrows):
    @functools.partial(
        pl.kernel, mesh=_sc_mesh(),
        out_type=jax.ShapeDtypeStruct((2, _NPAD, 128), _F32),
        scratch_types=[pltpu.VMEM((_KCH,), jnp.int32),
                       pltpu.VMEM((_KCH, 128), _F32),
                       pltpu.VMEM_SHARED((_NPAD, 128), _F32),
                       pltpu.SemaphoreType.DMA],
    )
    def k(m_hbm, d_hbm, z_hbm, agg_hbm, iv, rv, acc, sem):
        c = lax.axis_index("c")
        s = lax.axis_index("s")

        @pl.when(s == 0)
        def _():
            pltpu.sync_copy(z_hbm, acc)

        plsc.subcore_barrier()

        @pl.loop(0, _CH_S)
        def _(ch):
            off = s * _PER_S + ch * _KCH
            pltpu.sync_copy(d_hbm.at[pl.ds(off, _KCH)], iv)
            pltpu.sync_copy(m_hbm.at[c].at[pl.ds(off, _KCH)], rv)
            pltpu.sync_copy(rv, acc.at[iv], add=True)

        plsc.subcore_barrier()
        rows = _NPAD // 16
        pltpu.sync_copy(acc.at[pl.ds(s * rows, rows)],
                        agg_hbm.at[c].at[pl.ds(s * rows, rows)])

    return k(msg2, dstp, zrows)


def _dot(a, b):
    return jnp.dot(a, b, preferred_element_type=_F32)


def _embed_body(xs_ref, ea_ref, we_ref, be_ref, wn_ref, bn_ref, eo_ref, msg_ref):
    t = pl.program_id(0)
    e = _dot(ea_ref[...], we_ref[...]) + be_ref[...]
    eo_ref[...] = e
    hsrc = xs_ref[...] * wn_ref[...] + bn_ref[...]
    msg = jnp.maximum(hsrc + e, 0.0)
    ids = lax.broadcasted_iota(jnp.int32, (_TE, 1), 0) + t * _TE
    msg = jnp.where(ids < _E, msg, 0.0)
    msg_ref[0] = msg[:, :128]
    msg_ref[1] = msg[:, 128:]


def _edge_mlp_body(tg_ref, qd_ref, e_ref, c_ref, w2_ref, be1_ref, be2_ref, msg_ref):
    t = pl.program_id(0)
    tg = tg_ref[...]
    hs = tg[:, :256]
    ps = tg[:, 256:]
    e = e_ref[...]
    tt = jnp.maximum(ps + qd_ref[...] + _dot(e, c_ref[...]) + be1_ref[...], 0.0)
    eu = _dot(tt, w2_ref[...]) + be2_ref[...]
    e1 = (e + eu) * 0.5
    msg = jnp.maximum(hs + e1, 0.0)
    ids = lax.broadcasted_iota(jnp.int32, (_TEM, 1), 0) + t * _TEM
    msg = jnp.where(ids < _E, msg, 0.0)
    msg_ref[0] = msg[:, :128]
    msg_ref[1] = msg[:, 128:]


def _z_from_x_body(x_ref, agg_ref, wn_ref, bn_ref, w1_ref, b1_ref, w2_ref,
                   b2_ref, z_ref, st_ref):
    h = x_ref[...] * wn_ref[...] + bn_ref[...]
    _z_common(h, agg_ref, w1_ref, b1_ref, w2_ref, b2_ref, z_ref, st_ref)


def _z_from_h_body(h_ref, agg_ref, w1_ref, b1_ref, w2_ref, b2_ref, z_ref, st_ref):
    _z_common(h_ref[...], agg_ref, w1_ref, b1_ref, w2_ref, b2_ref, z_ref, st_ref)


def _z_common(h, agg_ref, w1_ref, b1_ref, w2_ref, b2_ref, z_ref, st_ref):
    agg = jnp.concatenate([agg_ref[0], agg_ref[1]], axis=1)
    zin = h + agg
    z1 = jnp.maximum(_dot(zin, w1_ref[...]) + b1_ref[...], 0.0)
    z = _dot(z1, w2_ref[...]) + b2_ref[...]
    z_ref[...] = z
    t = pl.program_id(0)

    @pl.when(t == 0)
    def _():
        st_ref[...] = jnp.zeros_like(st_ref)

    st_ref[0:1, :] += jnp.sum(z, axis=0, keepdims=True)
    st_ref[1:2, :] += jnp.sum(z * z, axis=0, keepdims=True)


def _bn_h(z_ref, st_ref, hprev):
    mu = st_ref[0:1, :] / _N
    var = st_ref[1:2, :] / _N - mu * mu
    zn = (z_ref[...] - mu) / jnp.sqrt(var + 1e-5)
    return (hprev + jnp.maximum(zn, 0.0)) * 0.5


def _ht_body(x_ref, z_ref, st_ref, wn_ref, bn_ref, a_ref, b_ref, t_ref, q_ref):
    h = x_ref[...] * wn_ref[...] + bn_ref[...]
    h1 = _bn_h(z_ref, st_ref, h)
    t_ref[:, :256] = h1
    t_ref[:, 256:] = _dot(h1, a_ref[...])
    q_ref[...] = _dot(h1, b_ref[...])


def _head_body(h1_ref, z_ref, st_ref, wc1_ref, bc1_ref, wc2_ref, bc2_ref, o_ref):
    h2 = _bn_h(z_ref, st_ref, h1_ref[...])
    y = jnp.maximum(_dot(h2, wc1_ref[...]) + bc1_ref[...], 0.0)
    o_ref[...] = _dot(y, wc2_ref[...]) + bc2_ref[...]


def _const(shape):
    return pl.BlockSpec(shape, lambda t: tuple(0 for _ in shape))


def _tc_embed(xsw, eap, W_edge, be, wn, bn):
    nt = _EPAD // _TE
    return pl.pallas_call(
        _embed_body,
        grid=(nt,),
        in_specs=[pl.BlockSpec((_TE, 1), lambda t: (t, 0)),
                  pl.BlockSpec((_TE, 16), lambda t: (t, 0)),
                  _const((16, 256)), _const((1, 256)),
                  _const((1, 256)), _const((1, 256))],
        out_specs=[pl.BlockSpec((_TE, 256), lambda t: (t, 0)),
                   pl.BlockSpec((2, _TE, 128), lambda t: (0, t, 0))],
        out_shape=[jax.ShapeDtypeStruct((_EPAD, 256), _F32),
                   jax.ShapeDtypeStruct((2, _EPAD, 128), _F32)],
    )(xsw, eap, W_edge, be, wn, bn)


def _tc_edge_mlp(Tg, Qd, e0, C, We2i, be1i, be2i):
    nt = _EPAD // _TEM
    return pl.pallas_call(
        _edge_mlp_body,
        grid=(nt,),
        in_specs=[pl.BlockSpec((_TEM, 512), lambda t: (t, 0)),
                  pl.BlockSpec((_TEM, 256), lambda t: (t, 0)),
                  pl.BlockSpec((_TEM, 256), lambda t: (t, 0)),
                  _const((256, 256)), _const((256, 256)),
                  _const((1, 256)), _const((1, 256))],
        out_specs=[pl.BlockSpec((2, _TEM, 128), lambda t: (0, t, 0))],
        out_shape=[jax.ShapeDtypeStruct((2, _EPAD, 128), _F32)],
    )(Tg, Qd, e0, C, We2i, be1i, be2i)[0]


def _tc_z(h_or_x, agg, wn_bn, w1, b1r, w2, b2r, from_x):
    if from_x:
        body = _z_from_x_body
        first_specs = [pl.BlockSpec((_TN, 1), lambda t: (t, 0))]
        first_args = [h_or_x]
        wspecs = [_const((1, 256)), _const((1, 256))]
        wargs = list(wn_bn)
    else:
        body = _z_from_h_body
        first_specs = [pl.BlockSpec((_TN, 256), lambda t: (t, 0))]
        first_args = [h_or_x]
        wspecs = []
        wargs = []
    return pl.pallas_call(
        body,
        grid=(_NT,),
        in_specs=first_specs
        + [pl.BlockSpec((2, _TN, 128), lambda t: (0, t, 0))]
        + wspecs
        + [_const((256, 256)), _const((1, 256)),
           _const((256, 256)), _const((1, 256))],
        out_specs=[pl.BlockSpec((_TN, 256), lambda t: (t, 0)),
                   _const((8, 256))],
        out_shape=[jax.ShapeDtypeStruct((_N, 256), _F32),
                   jax.ShapeDtypeStruct((8, 256), _F32)],
    )(*first_args, agg, *wargs, w1, b1r, w2, b2r)


def _tc_ht(x, z0, st0, wn, bn, A, B):
    return pl.pallas_call(
        _ht_body,
        grid=(_NT,),
        in_specs=[pl.BlockSpec((_TN, 1), lambda t: (t, 0)),
                  pl.BlockSpec((_TN, 256), lambda t: (t, 0)),
                  _const((8, 256)),
                  _const((1, 256)), _const((1, 256)),
                  _const((256, 256)), _const((256, 256))],
        out_specs=[pl.BlockSpec((_TN, 512), lambda t: (t, 0)),
                   pl.BlockSpec((_TN, 256), lambda t: (t, 0))],
        out_shape=[jax.ShapeDtypeStruct((_N, 512), _F32),
                   jax.ShapeDtypeStruct((_N, 256), _F32)],
    )(x, z0, st0, wn, bn, A, B)


def _tc_head(T, z1, st1, Wc1, bc1r, Wc2p, bc2r):
    return pl.pallas_call(
        _head_body,
        grid=(_NT,),
        in_specs=[pl.BlockSpec((_TN, 256), lambda t: (t, 0)),
                  pl.BlockSpec((_TN, 256), lambda t: (t, 0)),
                  _const((8, 256)),
                  _const((256, 256)), _const((1, 256)),
                  _const((256, 128)), _const((1, 128))],
        out_specs=[pl.BlockSpec((_TN, 128), lambda t: (t, 0))],
        out_shape=[jax.ShapeDtypeStruct((_N, 128), _F32)],
    )(T, z1, st1, Wc1, bc1r, Wc2p, bc2r)[0]


def kernel(x, edge_index, edge_attr, W_node, b_node, W_edge, b_edge, W1, b1,
           W2, b2, We1, be1, We2, be2, Wc1, bc1, Wc2, bc2):
    pad = _EPAD - _E
    padidx = jnp.arange(pad, dtype=jnp.int32) % _N
    srcp = jnp.concatenate([edge_index[0].astype(jnp.int32), padidx])
    dstp = jnp.concatenate([edge_index[1].astype(jnp.int32), padidx])
    eap = jnp.pad(edge_attr, ((0, pad), (0, 0)))
    zrows = jnp.zeros((_NPAD, 128), _F32)

    wn = W_node
    bn = b_node[None, :]
    ber = b_edge[None, :]
    A0, B0, C0 = We1[0, 0:256], We1[0, 256:512], We1[0, 512:768]
    Wc2p = jnp.pad(Wc2, ((0, 0), (0, 128 - Wc2.shape[1])))
    bc2r = jnp.pad(bc2, (0, 128 - bc2.shape[0]))[None, :]

    xs = _sc_gather_x(x[:, 0], srcp)
    e0, msg0 = _tc_embed(xs[:, None], eap, W_edge, ber, wn, bn)
    agg0 = _sc_scatter_add(msg0, dstp, zrows)
    z0, st0 = _tc_z(x, agg0, (wn, bn), W1[0], b1[0][None], W2[0], b2[0][None],
                    from_x=True)
    T, Q = _tc_ht(x, z0, st0, wn, bn, A0, B0)
    Tg, Qd = _sc_gather_tq(T, Q, srcp, dstp)
    msg1 = _tc_edge_mlp(Tg, Qd, e0, C0, We2[0], be1[0][None], be2[0][None])

    agg1 = _sc_scatter_add(msg1, dstp, zrows)
    z1, st1 = _tc_z(T, agg1, None, W1[1], b1[1][None], W2[1], b2[1][None],
                    from_x=False)
    outp = _tc_head(T, z1, st1, Wc1, bc1[None], Wc2p, bc2r)
    return outp[:, :Wc2.shape[1]]


# speedup vs baseline: 2.8331x; 1.2073x over previous
import functools

import jax
import jax.numpy as jnp
from jax import lax
from jax.experimental import pallas as pl
from jax.experimental.pallas import tpu as pltpu
from jax.experimental.pallas import tpu_sc as plsc

_N = 10000
_E = 160000
_H = 256
_EPAD = 163840
_KCH = 128
_NW = 32
_PER_W = _EPAD // _NW
_CH_W = _PER_W // _KCH
_PER_S = _EPAD // 16
_CH_S = _PER_S // _KCH
_NPAD = 10240
_TN = 1000
_NT = _N // _TN
_TE = 2048
_TEM = 1024
_F32 = jnp.float32


def _sc_mesh():
    return plsc.VectorSubcoreMesh(core_axis_name="c", subcore_axis_name="s")


def _sc_gather_x(xf, srcp):
    import dataclasses
    cp = pltpu.CompilerParams()
    if "needs_layout_passes" in pltpu.CompilerParams.__dataclass_fields__:
        cp = dataclasses.replace(cp, needs_layout_passes=False)

    @functools.partial(
        pl.kernel, mesh=_sc_mesh(),
        out_type=jax.ShapeDtypeStruct((_EPAD,), _F32),
        compiler_params=cp,
        scratch_types=[pltpu.VMEM((_N,), _F32),
                       pltpu.VMEM((_KCH,), jnp.int32),
                       pltpu.VMEM((_KCH,), _F32),
                       pltpu.SemaphoreType.DMA],
    )
    def k(x_hbm, s_hbm, o_hbm, xv, iv, ov, sem):
        w = lax.axis_index("s") * 2 + lax.axis_index("c")
        base = w * _PER_W
        pltpu.sync_copy(x_hbm, xv)

        @pl.loop(0, _CH_W)
        def _(ch):
            off = base + ch * _KCH
            pltpu.sync_copy(s_hbm.at[pl.ds(off, _KCH)], iv)

            @pl.loop(0, _KCH // 16)
            def _(j):
                idx = iv[pl.ds(j * 16, 16)]
                ov[pl.ds(j * 16, 16)] = plsc.load_gather(xv, [idx])

            pltpu.sync_copy(ov, o_hbm.at[pl.ds(off, _KCH)])

    return k(xf, srcp)


_KG = 64
_NCHG = _PER_W // _KG


def _sc_gather_tq(T, Q, srcp, dstp):
    @functools.partial(
        pl.kernel, mesh=_sc_mesh(),
        out_type=[jax.ShapeDtypeStruct((_EPAD, 512), _F32),
                  jax.ShapeDtypeStruct((_EPAD, 256), _F32)],
        scratch_types=[pltpu.VMEM((_PER_W,), jnp.int32),
                       pltpu.VMEM((_PER_W,), jnp.int32),
                       pltpu.VMEM((_KG, 512), _F32),
                       pltpu.VMEM((_KG, 512), _F32),
                       pltpu.VMEM((_KG, 256), _F32),
                       pltpu.VMEM((_KG, 256), _F32),
                       pltpu.SemaphoreType.DMA, pltpu.SemaphoreType.DMA,
                       pltpu.SemaphoreType.DMA, pltpu.SemaphoreType.DMA,
                       pltpu.SemaphoreType.DMA, pltpu.SemaphoreType.DMA,
                       pltpu.SemaphoreType.DMA, pltpu.SemaphoreType.DMA],
    )
    def k(t_hbm, q_hbm, s_hbm, d_hbm, tg_hbm, qd_hbm, ivs, ivd,
          rt0, rt1, rq0, rq1, sgt0, sgt1, sgq0, sgq1, swt0, swt1, swq0, swq1):
        w = lax.axis_index("s") * 2 + lax.axis_index("c")
        base = w * _PER_W
        pltpu.sync_copy(s_hbm.at[pl.ds(base, _PER_W)], ivs)
        pltpu.sync_copy(d_hbm.at[pl.ds(base, _PER_W)], ivd)

        def gstart(i, rt, rq, st, sq):
            pltpu.async_copy(t_hbm.at[ivs.at[pl.ds(i * _KG, _KG)]], rt, st)
            pltpu.async_copy(q_hbm.at[ivd.at[pl.ds(i * _KG, _KG)]], rq, sq)

        def gwait(i, rt, rq, st, sq):
            pltpu.make_async_copy(t_hbm.at[ivs.at[pl.ds(i * _KG, _KG)]], rt, st).wait()
            pltpu.make_async_copy(q_hbm.at[ivd.at[pl.ds(i * _KG, _KG)]], rq, sq).wait()

        def wstart(i, rt, rq, st, sq):
            pltpu.async_copy(rt, tg_hbm.at[pl.ds(base + i * _KG, _KG)], st)
            pltpu.async_copy(rq, qd_hbm.at[pl.ds(base + i * _KG, _KG)], sq)

        def wwait(i, rt, rq, st, sq):
            pltpu.make_async_copy(rt, tg_hbm.at[pl.ds(base + i * _KG, _KG)], st).wait()
            pltpu.make_async_copy(rq, qd_hbm.at[pl.ds(base + i * _KG, _KG)], sq).wait()

        gstart(0, rt0, rq0, sgt0, sgq0)
        gstart(1, rt1, rq1, sgt1, sgq1)

        @pl.loop(0, _NCHG // 2 - 1)
        def _(p):
            i0 = 2 * p
            gwait(i0, rt0, rq0, sgt0, sgq0)
            wstart(i0, rt0, rq0, swt0, swq0)
            gwait(i0 + 1, rt1, rq1, sgt1, sgq1)
            wstart(i0 + 1, rt1, rq1, swt1, swq1)
            wwait(i0, rt0, rq0, swt0, swq0)
            gstart(i0 + 2, rt0, rq0, sgt0, sgq0)
            wwait(i0 + 1, rt1, rq1, swt1, swq1)
            gstart(i0 + 3, rt1, rq1, sgt1, sgq1)

        i0 = _NCHG - 2
        gwait(i0, rt0, rq0, sgt0, sgq0)
        wstart(i0, rt0, rq0, swt0, swq0)
        gwait(i0 + 1, rt1, rq1, sgt1, sgq1)
        wstart(i0 + 1, rt1, rq1, swt1, swq1)
        wwait(i0, rt0, rq0, swt0, swq0)
        wwait(i0 + 1, rt1, rq1, swt1, swq1)

    return k(T, Q, srcp, dstp)


def _sc_scatter_add(msg2, dstp, zrows):
    @functools.partial(
        pl.kernel, mesh=_sc_mesh(),
        out_type=jax.ShapeDtypeStruct((2, _NPAD, 128), _F32),
        scratch_types=[pltpu.VMEM((_CH_S, _KCH), jnp.int32),
                       pltpu.VMEM((_KCH, 128), _F32),
                       pltpu.VMEM((_KCH, 128), _F32),
                       pltpu.VMEM_SHARED((_NPAD, 128), _F32),
                       pltpu.SemaphoreType.DMA, pltpu.SemaphoreType.DMA],
    )
    def k(m_hbm, d3_hbm, z_hbm, agg_hbm, iv2d, rv0, rv1, acc, sm0, sm1):
        c = lax.axis_index("c")
        s = lax.axis_index("s")

        @pl.when(s == 0)
        def _():
            pltpu.sync_copy(z_hbm, acc)

        pltpu.sync_copy(d3_hbm.at[s], iv2d)
        plsc.subcore_barrier()
        mbase = s * _PER_S

        def mstart(i, rv, sm):
            pltpu.async_copy(m_hbm.at[c].at[pl.ds(mbase + i * _KCH, _KCH)], rv, sm)

        def mwait(i, rv, sm):
            pltpu.make_async_copy(
                m_hbm.at[c].at[pl.ds(mbase + i * _KCH, _KCH)], rv, sm).wait()

        def addi(i, rv):
            pltpu.sync_copy(rv, acc.at[iv2d.at[i]], add=True)

        mstart(0, rv0, sm0)
        mstart(1, rv1, sm1)

        @pl.loop(0, _CH_S // 2 - 1)
        def _(p):
            i0 = 2 * p
            mwait(i0, rv0, sm0)
            addi(i0, rv0)
            mstart(i0 + 2, rv0, sm0)
            mwait(i0 + 1, rv1, sm1)
            addi(i0 + 1, rv1)
            mstart(i0 + 3, rv1, sm1)

        i0 = _CH_S - 2
        mwait(i0, rv0, sm0)
        addi(i0, rv0)
        mwait(i0 + 1, rv1, sm1)
        addi(i0 + 1, rv1)

        plsc.subcore_barrier()
        rows = _NPAD // 16
        pltpu.sync_copy(acc.at[pl.ds(s * rows, rows)],
                        agg_hbm.at[c].at[pl.ds(s * rows, rows)])

    return k(msg2, dstp, zrows)


def _dot(a, b):
    return jnp.dot(a, b, preferred_element_type=_F32)


def _embed_body(xs_ref, ea_ref, we_ref, be_ref, wn_ref, bn_ref, eo_ref, msg_ref):
    t = pl.program_id(0)
    e = _dot(ea_ref[...], we_ref[...]) + be_ref[...]
    eo_ref[...] = e
    hsrc = xs_ref[...] * wn_ref[...] + bn_ref[...]
    msg = jnp.maximum(hsrc + e, 0.0)
    ids = lax.broadcasted_iota(jnp.int32, (_TE, 1), 0) + t * _TE
    msg = jnp.where(ids < _E, msg, 0.0)
    msg_ref[0] = msg[:, :128]
    msg_ref[1] = msg[:, 128:]


def _edge_mlp_body(tg_ref, qd_ref, e_ref, c_ref, w2_ref, be1_ref, be2_ref, msg_ref):
    t = pl.program_id(0)
    tg = tg_ref[...]
    hs = tg[:, :256]
    ps = tg[:, 256:]
    e = e_ref[...]
    tt = jnp.maximum(ps + qd_ref[...] + _dot(e, c_ref[...]) + be1_ref[...], 0.0)
    eu = _dot(tt, w2_ref[...]) + be2_ref[...]
    e1 = (e + eu) * 0.5
    msg = jnp.maximum(hs + e1, 0.0)
    ids = lax.broadcasted_iota(jnp.int32, (_TEM, 1), 0) + t * _TEM
    msg = jnp.where(ids < _E, msg, 0.0)
    msg_ref[0] = msg[:, :128]
    msg_ref[1] = msg[:, 128:]


def _z_from_x_body(x_ref, agg_ref, wn_ref, bn_ref, w1_ref, b1_ref, w2_ref,
                   b2_ref, z_ref, st_ref):
    h = x_ref[...] * wn_ref[...] + bn_ref[...]
    _z_common(h, agg_ref, w1_ref, b1_ref, w2_ref, b2_ref, z_ref, st_ref)


def _z_from_h_body(h_ref, agg_ref, w1_ref, b1_ref, w2_ref, b2_ref, z_ref, st_ref):
    _z_common(h_ref[...], agg_ref, w1_ref, b1_ref, w2_ref, b2_ref, z_ref, st_ref)


def _z_common(h, agg_ref, w1_ref, b1_ref, w2_ref, b2_ref, z_ref, st_ref):
    agg = jnp.concatenate([agg_ref[0], agg_ref[1]], axis=1)
    zin = h + agg
    z1 = jnp.maximum(_dot(zin, w1_ref[...]) + b1_ref[...], 0.0)
    z = _dot(z1, w2_ref[...]) + b2_ref[...]
    z_ref[...] = z
    t = pl.program_id(0)

    @pl.when(t == 0)
    def _():
        st_ref[...] = jnp.zeros_like(st_ref)

    st_ref[0:1, :] += jnp.sum(z, axis=0, keepdims=True)
    st_ref[1:2, :] += jnp.sum(z * z, axis=0, keepdims=True)


def _bn_h(z_ref, st_ref, hprev):
    mu = st_ref[0:1, :] / _N
    var = st_ref[1:2, :] / _N - mu * mu
    zn = (z_ref[...] - mu) / jnp.sqrt(var + 1e-5)
    return (hprev + jnp.maximum(zn, 0.0)) * 0.5


def _ht_body(x_ref, z_ref, st_ref, wn_ref, bn_ref, a_ref, b_ref, t_ref, q_ref):
    h = x_ref[...] * wn_ref[...] + bn_ref[...]
    h1 = _bn_h(z_ref, st_ref, h)
    t_ref[:, :256] = h1
    t_ref[:, 256:] = _dot(h1, a_ref[...])
    q_ref[...] = _dot(h1, b_ref[...])


def _head_body(h1_ref, z_ref, st_ref, wc1_ref, bc1_ref, wc2_ref, bc2_ref, o_ref):
    h2 = _bn_h(z_ref, st_ref, h1_ref[...])
    y = jnp.maximum(_dot(h2, wc1_ref[...]) + bc1_ref[...], 0.0)
    o_ref[...] = _dot(y, wc2_ref[...]) + bc2_ref[...]


def _const(shape):
    return pl.BlockSpec(shape, lambda t: tuple(0 for _ in shape))


def _tc_embed(xsw, eap, W_edge, be, wn, bn):
    nt = _EPAD // _TE
    return pl.pallas_call(
        _embed_body,
        grid=(nt,),
        in_specs=[pl.BlockSpec((_TE, 1), lambda t: (t, 0)),
                  pl.BlockSpec((_TE, 16), lambda t: (t, 0)),
                  _const((16, 256)), _const((1, 256)),
                  _const((1, 256)), _const((1, 256))],
        out_specs=[pl.BlockSpec((_TE, 256), lambda t: (t, 0)),
                   pl.BlockSpec((2, _TE, 128), lambda t: (0, t, 0))],
        out_shape=[jax.ShapeDtypeStruct((_EPAD, 256), _F32),
                   jax.ShapeDtypeStruct((2, _EPAD, 128), _F32)],
    )(xsw, eap, W_edge, be, wn, bn)


def _tc_edge_mlp(Tg, Qd, e0, C, We2i, be1i, be2i):
    nt = _EPAD // _TEM
    return pl.pallas_call(
        _edge_mlp_body,
        grid=(nt,),
        in_specs=[pl.BlockSpec((_TEM, 512), lambda t: (t, 0)),
                  pl.BlockSpec((_TEM, 256), lambda t: (t, 0)),
                  pl.BlockSpec((_TEM, 256), lambda t: (t, 0)),
                  _const((256, 256)), _const((256, 256)),
                  _const((1, 256)), _const((1, 256))],
        out_specs=[pl.BlockSpec((2, _TEM, 128), lambda t: (0, t, 0))],
        out_shape=[jax.ShapeDtypeStruct((2, _EPAD, 128), _F32)],
    )(Tg, Qd, e0, C, We2i, be1i, be2i)[0]


def _tc_z(h_or_x, agg, wn_bn, w1, b1r, w2, b2r, from_x):
    if from_x:
        body = _z_from_x_body
        first_specs = [pl.BlockSpec((_TN, 1), lambda t: (t, 0))]
        first_args = [h_or_x]
        wspecs = [_const((1, 256)), _const((1, 256))]
        wargs = list(wn_bn)
    else:
        body = _z_from_h_body
        first_specs = [pl.BlockSpec((_TN, 256), lambda t: (t, 0))]
        first_args = [h_or_x]
        wspecs = []
        wargs = []
    return pl.pallas_call(
        body,
        grid=(_NT,),
        in_specs=first_specs
        + [pl.BlockSpec((2, _TN, 128), lambda t: (0, t, 0))]
        + wspecs
        + [_const((256, 256)), _const((1, 256)),
           _const((256, 256)), _const((1, 256))],
        out_specs=[pl.BlockSpec((_TN, 256), lambda t: (t, 0)),
                   _const((8, 256))],
        out_shape=[jax.ShapeDtypeStruct((_N, 256), _F32),
                   jax.ShapeDtypeStruct((8, 256), _F32)],
    )(*first_args, agg, *wargs, w1, b1r, w2, b2r)


def _tc_ht(x, z0, st0, wn, bn, A, B):
    return pl.pallas_call(
        _ht_body,
        grid=(_NT,),
        in_specs=[pl.BlockSpec((_TN, 1), lambda t: (t, 0)),
                  pl.BlockSpec((_TN, 256), lambda t: (t, 0)),
                  _const((8, 256)),
                  _const((1, 256)), _const((1, 256)),
                  _const((256, 256)), _const((256, 256))],
        out_specs=[pl.BlockSpec((_TN, 512), lambda t: (t, 0)),
                   pl.BlockSpec((_TN, 256), lambda t: (t, 0))],
        out_shape=[jax.ShapeDtypeStruct((_N, 512), _F32),
                   jax.ShapeDtypeStruct((_N, 256), _F32)],
    )(x, z0, st0, wn, bn, A, B)


def _tc_head(T, z1, st1, Wc1, bc1r, Wc2p, bc2r):
    return pl.pallas_call(
        _head_body,
        grid=(_NT,),
        in_specs=[pl.BlockSpec((_TN, 256), lambda t: (t, 0)),
                  pl.BlockSpec((_TN, 256), lambda t: (t, 0)),
                  _const((8, 256)),
                  _const((256, 256)), _const((1, 256)),
                  _const((256, 128)), _const((1, 128))],
        out_specs=[pl.BlockSpec((_TN, 128), lambda t: (t, 0))],
        out_shape=[jax.ShapeDtypeStruct((_N, 128), _F32)],
    )(T, z1, st1, Wc1, bc1r, Wc2p, bc2r)[0]


def kernel(x, edge_index, edge_attr, W_node, b_node, W_edge, b_edge, W1, b1,
           W2, b2, We1, be1, We2, be2, Wc1, bc1, Wc2, bc2):
    pad = _EPAD - _E
    padidx = jnp.arange(pad, dtype=jnp.int32) % _N
    srcp = jnp.concatenate([edge_index[0].astype(jnp.int32), padidx])
    dstp = jnp.concatenate([edge_index[1].astype(jnp.int32), padidx])
    eap = jnp.pad(edge_attr, ((0, pad), (0, 0)))
    dst3 = dstp.reshape(16, _CH_S, _KCH)
    zrows = jnp.zeros((_NPAD, 128), _F32)

    wn = W_node
    bn = b_node[None, :]
    ber = b_edge[None, :]
    A0, B0, C0 = We1[0, 0:256], We1[0, 256:512], We1[0, 512:768]
    Wc2p = jnp.pad(Wc2, ((0, 0), (0, 128 - Wc2.shape[1])))
    bc2r = jnp.pad(bc2, (0, 128 - bc2.shape[0]))[None, :]

    xs = _sc_gather_x(x[:, 0], srcp)
    e0, msg0 = _tc_embed(xs[:, None], eap, W_edge, ber, wn, bn)
    agg0 = _sc_scatter_add(msg0, dst3, zrows)
    z0, st0 = _tc_z(x, agg0, (wn, bn), W1[0], b1[0][None], W2[0], b2[0][None],
                    from_x=True)
    T, Q = _tc_ht(x, z0, st0, wn, bn, A0, B0)
    Tg, Qd = _sc_gather_tq(T, Q, srcp, dstp)
    msg1 = _tc_edge_mlp(Tg, Qd, e0, C0, We2[0], be1[0][None], be2[0][None])

    agg1 = _sc_scatter_add(msg1, dst3, zrows)
    z1, st1 = _tc_z(T, agg1, None, W1[1], b1[1][None], W2[1], b2[1][None],
                    from_x=False)
    outp = _tc_head(T, z1, st1, Wc1, bc1[None], Wc2p, bc2r)
    return outp[:, :Wc2.shape[1]]


# speedup vs baseline: 3.1740x; 1.1203x over previous
import functools

import jax
import jax.numpy as jnp
from jax import lax
from jax.experimental import pallas as pl
from jax.experimental.pallas import tpu as pltpu
from jax.experimental.pallas import tpu_sc as plsc

_N = 10000
_E = 160000
_H = 256
_EPAD = 163840
_KCH = 128
_NW = 32
_PER_W = _EPAD // _NW
_PER_S = _EPAD // 16
_CH_S = _PER_S // _KCH
_KG = 64
_NCHG = _PER_W // _KG
_NPAD = 10240
_TN = 1000
_NT = _N // _TN
_TE = 2048
_TEM = 1024
_F32 = jnp.float32


def _sc_mesh():
    return plsc.VectorSubcoreMesh(core_axis_name="c", subcore_axis_name="s")


def _sc_gather_x(xf, srcp):
    import dataclasses
    cp = pltpu.CompilerParams()
    if "needs_layout_passes" in pltpu.CompilerParams.__dataclass_fields__:
        cp = dataclasses.replace(cp, needs_layout_passes=False)

    @functools.partial(
        pl.kernel, mesh=_sc_mesh(),
        out_type=jax.ShapeDtypeStruct((_EPAD,), _F32),
        compiler_params=cp,
        scratch_types=[pltpu.VMEM((_N,), _F32),
                       pltpu.VMEM((_PER_W,), jnp.int32),
                       pltpu.VMEM((_PER_W,), _F32),
                       pltpu.SemaphoreType.DMA],
    )
    def k(x_hbm, s_hbm, o_hbm, xv, iv, ov, sem):
        w = lax.axis_index("s") * 2 + lax.axis_index("c")
        base = w * _PER_W
        pltpu.sync_copy(x_hbm, xv)
        pltpu.sync_copy(s_hbm.at[pl.ds(base, _PER_W)], iv)

        @pl.loop(0, _PER_W // 16)
        def _(j):
            idx = iv[pl.ds(j * 16, 16)]
            ov[pl.ds(j * 16, 16)] = plsc.load_gather(xv, [idx])

        pltpu.sync_copy(ov, o_hbm.at[pl.ds(base, _PER_W)])

    return k(xf, srcp)


def _sc_gather_hh(H, srcp, dstp):
    @functools.partial(
        pl.kernel, mesh=_sc_mesh(),
        out_type=[jax.ShapeDtypeStruct((_EPAD, 256), _F32),
                  jax.ShapeDtypeStruct((_EPAD, 256), _F32)],
        scratch_types=[pltpu.VMEM((_PER_W,), jnp.int32),
                       pltpu.VMEM((_PER_W,), jnp.int32),
                       pltpu.VMEM((_KG, 256), _F32),
                       pltpu.VMEM((_KG, 256), _F32),
                       pltpu.VMEM((_KG, 256), _F32),
                       pltpu.VMEM((_KG, 256), _F32),
                       pltpu.SemaphoreType.DMA, pltpu.SemaphoreType.DMA,
                       pltpu.SemaphoreType.DMA, pltpu.SemaphoreType.DMA,
                       pltpu.SemaphoreType.DMA, pltpu.SemaphoreType.DMA,
                       pltpu.SemaphoreType.DMA, pltpu.SemaphoreType.DMA],
    )
    def k(h_hbm, s_hbm, d_hbm, hs_hbm, hd_hbm, ivs, ivd,
          rt0, rt1, rq0, rq1, sgt0, sgt1, sgq0, sgq1, swt0, swt1, swq0, swq1):
        w = lax.axis_index("s") * 2 + lax.axis_index("c")
        base = w * _PER_W
        pltpu.sync_copy(s_hbm.at[pl.ds(base, _PER_W)], ivs)
        pltpu.sync_copy(d_hbm.at[pl.ds(base, _PER_W)], ivd)

        def gstart(i, rt, rq, st, sq):
            pltpu.async_copy(h_hbm.at[ivs.at[pl.ds(i * _KG, _KG)]], rt, st)
            pltpu.async_copy(h_hbm.at[ivd.at[pl.ds(i * _KG, _KG)]], rq, sq)

        def gwait(i, rt, rq, st, sq):
            pltpu.make_async_copy(h_hbm.at[ivs.at[pl.ds(i * _KG, _KG)]], rt, st).wait()
            pltpu.make_async_copy(h_hbm.at[ivd.at[pl.ds(i * _KG, _KG)]], rq, sq).wait()

        def wstart(i, rt, rq, st, sq):
            pltpu.async_copy(rt, hs_hbm.at[pl.ds(base + i * _KG, _KG)], st)
            pltpu.async_copy(rq, hd_hbm.at[pl.ds(base + i * _KG, _KG)], sq)

        def wwait(i, rt, rq, st, sq):
            pltpu.make_async_copy(rt, hs_hbm.at[pl.ds(base + i * _KG, _KG)], st).wait()
            pltpu.make_async_copy(rq, hd_hbm.at[pl.ds(base + i * _KG, _KG)], sq).wait()

        gstart(0, rt0, rq0, sgt0, sgq0)
        gstart(1, rt1, rq1, sgt1, sgq1)

        @pl.loop(0, _NCHG // 2 - 1)
        def _(p):
            i0 = 2 * p
            gwait(i0, rt0, rq0, sgt0, sgq0)
            wstart(i0, rt0, rq0, swt0, swq0)
            gwait(i0 + 1, rt1, rq1, sgt1, sgq1)
            wstart(i0 + 1, rt1, rq1, swt1, swq1)
            wwait(i0, rt0, rq0, swt0, swq0)
            gstart(i0 + 2, rt0, rq0, sgt0, sgq0)
            wwait(i0 + 1, rt1, rq1, swt1, swq1)
            gstart(i0 + 3, rt1, rq1, sgt1, sgq1)

        i0 = _NCHG - 2
        gwait(i0, rt0, rq0, sgt0, sgq0)
        wstart(i0, rt0, rq0, swt0, swq0)
        gwait(i0 + 1, rt1, rq1, sgt1, sgq1)
        wstart(i0 + 1, rt1, rq1, swt1, swq1)
        wwait(i0, rt0, rq0, swt0, swq0)
        wwait(i0 + 1, rt1, rq1, swt1, swq1)

    return k(H, srcp, dstp)


def _sc_scatter_add(msg2, dst3, zrows):
    @functools.partial(
        pl.kernel, mesh=_sc_mesh(),
        out_type=jax.ShapeDtypeStruct((2, _NPAD, 128), _F32),
        scratch_types=[pltpu.VMEM((_CH_S, _KCH), jnp.int32),
                       pltpu.VMEM((_KCH, 128), _F32),
                       pltpu.VMEM((_KCH, 128), _F32),
                       pltpu.VMEM_SHARED((_NPAD, 128), _F32),
                       pltpu.SemaphoreType.DMA, pltpu.SemaphoreType.DMA],
    )
    def k(m_hbm, d3_hbm, z_hbm, agg_hbm, iv2d, rv0, rv1, acc, sm0, sm1):
        c = lax.axis_index("c")
        s = lax.axis_index("s")

        @pl.when(s == 0)
        def _():
            pltpu.sync_copy(z_hbm, acc)

        pltpu.sync_copy(d3_hbm.at[s], iv2d)
        plsc.subcore_barrier()
        mbase = s * _PER_S

        def mstart(i, rv, sm):
            pltpu.async_copy(m_hbm.at[c].at[pl.ds(mbase + i * _KCH, _KCH)], rv, sm)

        def mwait(i, rv, sm):
            pltpu.make_async_copy(
                m_hbm.at[c].at[pl.ds(mbase + i * _KCH, _KCH)], rv, sm).wait()

        def addi(i, rv):
            pltpu.sync_copy(rv, acc.at[iv2d.at[i]], add=True)

        mstart(0, rv0, sm0)
        mstart(1, rv1, sm1)

        @pl.loop(0, _CH_S // 2 - 1)
        def _(p):
            i0 = 2 * p
            mwait(i0, rv0, sm0)
            addi(i0, rv0)
            mstart(i0 + 2, rv0, sm0)
            mwait(i0 + 1, rv1, sm1)
            addi(i0 + 1, rv1)
            mstart(i0 + 3, rv1, sm1)

        i0 = _CH_S - 2
        mwait(i0, rv0, sm0)
        addi(i0, rv0)
        mwait(i0 + 1, rv1, sm1)
        addi(i0 + 1, rv1)

        plsc.subcore_barrier()
        rows = _NPAD // 16
        pltpu.sync_copy(acc.at[pl.ds(s * rows, rows)],
                        agg_hbm.at[c].at[pl.ds(s * rows, rows)])

    return k(msg2, dst3, zrows)


def _dot(a, b):
    return jnp.dot(a, b, preferred_element_type=_F32)


def _embed_body(xs_ref, ea_ref, we_ref, be_ref, wn_ref, bn_ref, eo_ref, msg_ref):
    t = pl.program_id(0)
    e = _dot(ea_ref[...], we_ref[...]) + be_ref[...]
    eo_ref[...] = e
    hsrc = xs_ref[...] * wn_ref[...] + bn_ref[...]
    msg = jnp.maximum(hsrc + e, 0.0)
    ids = lax.broadcasted_iota(jnp.int32, (_TE, 1), 0) + t * _TE
    msg = jnp.where(ids < _E, msg, 0.0)
    msg_ref[0] = msg[:, :128]
    msg_ref[1] = msg[:, 128:]


def _edge_mlp_body(hs_ref, hd_ref, e_ref, a_ref, b_ref, c_ref, w2_ref,
                   be1_ref, be2_ref, msg_ref):
    t = pl.program_id(0)
    hs = hs_ref[...]
    e = e_ref[...]
    tt = jnp.maximum(
        _dot(hs, a_ref[...]) + _dot(hd_ref[...], b_ref[...])
        + _dot(e, c_ref[...]) + be1_ref[...], 0.0)
    eu = _dot(tt, w2_ref[...]) + be2_ref[...]
    e1 = (e + eu) * 0.5
    msg = jnp.maximum(hs + e1, 0.0)
    ids = lax.broadcasted_iota(jnp.int32, (_TEM, 1), 0) + t * _TEM
    msg = jnp.where(ids < _E, msg, 0.0)
    msg_ref[0] = msg[:, :128]
    msg_ref[1] = msg[:, 128:]


def _z_from_x_body(x_ref, agg_ref, wn_ref, bn_ref, w1_ref, b1_ref, w2_ref,
                   b2_ref, z_ref, st_ref):
    h = x_ref[...] * wn_ref[...] + bn_ref[...]
    _z_common(h, agg_ref, w1_ref, b1_ref, w2_ref, b2_ref, z_ref, st_ref)


def _z_from_h_body(h_ref, agg_ref, w1_ref, b1_ref, w2_ref, b2_ref, z_ref, st_ref):
    _z_common(h_ref[...], agg_ref, w1_ref, b1_ref, w2_ref, b2_ref, z_ref, st_ref)


def _z_common(h, agg_ref, w1_ref, b1_ref, w2_ref, b2_ref, z_ref, st_ref):
    agg = jnp.concatenate([agg_ref[0], agg_ref[1]], axis=1)
    zin = h + agg
    z1 = jnp.maximum(_dot(zin, w1_ref[...]) + b1_ref[...], 0.0)
    z = _dot(z1, w2_ref[...]) + b2_ref[...]
    z_ref[...] = z
    t = pl.program_id(0)

    @pl.when(t == 0)
    def _():
        st_ref[...] = jnp.zeros_like(st_ref)

    st_ref[0:1, :] += jnp.sum(z, axis=0, keepdims=True)
    st_ref[1:2, :] += jnp.sum(z * z, axis=0, keepdims=True)


def _bn_h(z_ref, st_ref, hprev):
    mu = st_ref[0:1, :] / _N
    var = st_ref[1:2, :] / _N - mu * mu
    zn = (z_ref[...] - mu) / jnp.sqrt(var + 1e-5)
    return (hprev + jnp.maximum(zn, 0.0)) * 0.5


def _ht_body(x_ref, z_ref, st_ref, wn_ref, bn_ref, h_ref):
    h = x_ref[...] * wn_ref[...] + bn_ref[...]
    h_ref[...] = _bn_h(z_ref, st_ref, h)


def _head_body(h1_ref, z_ref, st_ref, wc1_ref, bc1_ref, wc2_ref, bc2_ref, o_ref):
    h2 = _bn_h(z_ref, st_ref, h1_ref[...])
    y = jnp.maximum(_dot(h2, wc1_ref[...]) + bc1_ref[...], 0.0)
    o_ref[...] = _dot(y, wc2_ref[...]) + bc2_ref[...]


def _const(shape):
    return pl.BlockSpec(shape, lambda t: tuple(0 for _ in shape))


def _tc_embed(xs, eap, W_edge, be, wn, bn):
    nt = _EPAD // _TE
    return pl.pallas_call(
        _embed_body,
        grid=(nt,),
        in_specs=[pl.BlockSpec((_TE, 1), lambda t: (t, 0)),
                  pl.BlockSpec((_TE, 16), lambda t: (t, 0)),
                  _const((16, 256)), _const((1, 256)),
                  _const((1, 256)), _const((1, 256))],
        out_specs=[pl.BlockSpec((_TE, 256), lambda t: (t, 0)),
                   pl.BlockSpec((2, _TE, 128), lambda t: (0, t, 0))],
        out_shape=[jax.ShapeDtypeStruct((_EPAD, 256), _F32),
                   jax.ShapeDtypeStruct((2, _EPAD, 128), _F32)],
    )(xs, eap, W_edge, be, wn, bn)


def _tc_edge_mlp(Hs, Hd, e0, A, B, C, We2i, be1i, be2i):
    nt = _EPAD // _TEM
    return pl.pallas_call(
        _edge_mlp_body,
        grid=(nt,),
        in_specs=[pl.BlockSpec((_TEM, 256), lambda t: (t, 0)),
                  pl.BlockSpec((_TEM, 256), lambda t: (t, 0)),
                  pl.BlockSpec((_TEM, 256), lambda t: (t, 0)),
                  _const((256, 256)), _const((256, 256)), _const((256, 256)),
                  _const((256, 256)),
                  _const((1, 256)), _const((1, 256))],
        out_specs=[pl.BlockSpec((2, _TEM, 128), lambda t: (0, t, 0))],
        out_shape=[jax.ShapeDtypeStruct((2, _EPAD, 128), _F32)],
    )(Hs, Hd, e0, A, B, C, We2i, be1i, be2i)[0]


def _tc_z(h_or_x, agg, wn_bn, w1, b1r, w2, b2r, from_x):
    if from_x:
        body = _z_from_x_body
        first_specs = [pl.BlockSpec((_TN, 1), lambda t: (t, 0))]
        wspecs = [_const((1, 256)), _const((1, 256))]
        wargs = list(wn_bn)
    else:
        body = _z_from_h_body
        first_specs = [pl.BlockSpec((_TN, 256), lambda t: (t, 0))]
        wspecs = []
        wargs = []
    return pl.pallas_call(
        body,
        grid=(_NT,),
        in_specs=first_specs
        + [pl.BlockSpec((2, _TN, 128), lambda t: (0, t, 0))]
        + wspecs
        + [_const((256, 256)), _const((1, 256)),
           _const((256, 256)), _const((1, 256))],
        out_specs=[pl.BlockSpec((_TN, 256), lambda t: (t, 0)),
                   _const((8, 256))],
        out_shape=[jax.ShapeDtypeStruct((_N, 256), _F32),
                   jax.ShapeDtypeStruct((8, 256), _F32)],
    )(h_or_x, agg, *wargs, w1, b1r, w2, b2r)


def _tc_ht(x, z0, st0, wn, bn):
    return pl.pallas_call(
        _ht_body,
        grid=(_NT,),
        in_specs=[pl.BlockSpec((_TN, 1), lambda t: (t, 0)),
                  pl.BlockSpec((_TN, 256), lambda t: (t, 0)),
                  _const((8, 256)),
                  _const((1, 256)), _const((1, 256))],
        out_specs=[pl.BlockSpec((_TN, 256), lambda t: (t, 0))],
        out_shape=[jax.ShapeDtypeStruct((_N, 256), _F32)],
    )(x, z0, st0, wn, bn)[0]


def _tc_head(H1, z1, st1, Wc1, bc1r, Wc2p, bc2r):
    return pl.pallas_call(
        _head_body,
        grid=(_NT,),
        in_specs=[pl.BlockSpec((_TN, 256), lambda t: (t, 0)),
                  pl.BlockSpec((_TN, 256), lambda t: (t, 0)),
                  _const((8, 256)),
                  _const((256, 256)), _const((1, 256)),
                  _const((256, 128)), _const((1, 128))],
        out_specs=[pl.BlockSpec((_TN, 128), lambda t: (t, 0))],
        out_shape=[jax.ShapeDtypeStruct((_N, 128), _F32)],
    )(H1, z1, st1, Wc1, bc1r, Wc2p, bc2r)[0]


def kernel(x, edge_index, edge_attr, W_node, b_node, W_edge, b_edge, W1, b1,
           W2, b2, We1, be1, We2, be2, Wc1, bc1, Wc2, bc2):
    pad = _EPAD - _E
    padidx = jnp.arange(pad, dtype=jnp.int32) % _N
    srcp = jnp.concatenate([edge_index[0].astype(jnp.int32), padidx])
    dstp = jnp.concatenate([edge_index[1].astype(jnp.int32), padidx])
    eap = jnp.pad(edge_attr, ((0, pad), (0, 0)))
    dst3 = dstp.reshape(16, _CH_S, _KCH)
    zrows = jnp.zeros((_NPAD, 128), _F32)

    wn = W_node
    bn = b_node[None, :]
    ber = b_edge[None, :]
    A0, B0, C0 = We1[0, 0:256], We1[0, 256:512], We1[0, 512:768]
    Wc2p = jnp.pad(Wc2, ((0, 0), (0, 128 - Wc2.shape[1])))
    bc2r = jnp.pad(bc2, (0, 128 - bc2.shape[0]))[None, :]

    xs = _sc_gather_x(x[:, 0], srcp)
    e0, msg0 = _tc_embed(xs[:, None], eap, W_edge, ber, wn, bn)
    agg0 = _sc_scatter_add(msg0, dst3, zrows)
    z0, st0 = _tc_z(x, agg0, (wn, bn), W1[0], b1[0][None], W2[0], b2[0][None],
                    from_x=True)
    H1 = _tc_ht(x, z0, st0, wn, bn)
    Hs, Hd = _sc_gather_hh(H1, srcp, dstp)
    msg1 = _tc_edge_mlp(Hs, Hd, e0, A0, B0, C0, We2[0],
                        be1[0][None], be2[0][None])

    agg1 = _sc_scatter_add(msg1, dst3, zrows)
    z1, st1 = _tc_z(H1, agg1, None, W1[1], b1[1][None], W2[1], b2[1][None],
                    from_x=False)
    outp = _tc_head(H1, z1, st1, Wc1, bc1[None], Wc2p, bc2r)
    return outp[:, :Wc2.shape[1]]


# speedup vs baseline: 3.2851x; 1.0350x over previous
import functools

import jax
import jax.numpy as jnp
from jax import lax
from jax.experimental import pallas as pl
from jax.experimental.pallas import tpu as pltpu
from jax.experimental.pallas import tpu_sc as plsc

_N = 10000
_E = 160000
_H = 256
_EPAD = 163840
_KCH = 128
_NW = 32
_PER_W = _EPAD // _NW
_PER_S = _EPAD // 16
_CH_S = _PER_S // _KCH
_KG = 64
_NCHG = _PER_W // _KG
_NPAD = 10240
_TN = 1000
_NT = _N // _TN
_TE = 2048
_TEM = 1024
_F32 = jnp.float32


def _sc_mesh():
    return plsc.VectorSubcoreMesh(core_axis_name="c", subcore_axis_name="s")


def _sc_gather_x(xf, srcp):
    import dataclasses
    cp = pltpu.CompilerParams()
    if "needs_layout_passes" in pltpu.CompilerParams.__dataclass_fields__:
        cp = dataclasses.replace(cp, needs_layout_passes=False)

    @functools.partial(
        pl.kernel, mesh=_sc_mesh(),
        out_type=jax.ShapeDtypeStruct((_EPAD,), _F32),
        compiler_params=cp,
        scratch_types=[pltpu.VMEM((_N,), _F32),
                       pltpu.VMEM((_PER_W,), jnp.int32),
                       pltpu.VMEM((_PER_W,), _F32),
                       pltpu.SemaphoreType.DMA],
    )
    def k(x_hbm, s_hbm, o_hbm, xv, iv, ov, sem):
        w = lax.axis_index("s") * 2 + lax.axis_index("c")
        base = w * _PER_W
        pltpu.sync_copy(x_hbm, xv)
        pltpu.sync_copy(s_hbm.at[pl.ds(base, _PER_W)], iv)

        @pl.loop(0, _PER_W // 16)
        def _(j):
            idx = iv[pl.ds(j * 16, 16)]
            ov[pl.ds(j * 16, 16)] = plsc.load_gather(xv, [idx])

        pltpu.sync_copy(ov, o_hbm.at[pl.ds(base, _PER_W)])

    return k(xf, srcp)


def _sc_gather_hh(H, srcp, dstp, start, count):
    per_w = count // _NW
    nchg = per_w // _KG

    @functools.partial(
        pl.kernel, mesh=_sc_mesh(),
        out_type=[jax.ShapeDtypeStruct((count, 256), _F32),
                  jax.ShapeDtypeStruct((count, 256), _F32)],
        scratch_types=[pltpu.VMEM((per_w,), jnp.int32),
                       pltpu.VMEM((per_w,), jnp.int32),
                       pltpu.VMEM((_KG, 256), _F32),
                       pltpu.VMEM((_KG, 256), _F32),
                       pltpu.VMEM((_KG, 256), _F32),
                       pltpu.VMEM((_KG, 256), _F32),
                       pltpu.SemaphoreType.DMA, pltpu.SemaphoreType.DMA,
                       pltpu.SemaphoreType.DMA, pltpu.SemaphoreType.DMA,
                       pltpu.SemaphoreType.DMA, pltpu.SemaphoreType.DMA,
                       pltpu.SemaphoreType.DMA, pltpu.SemaphoreType.DMA],
    )
    def k(h_hbm, s_hbm, d_hbm, hs_hbm, hd_hbm, ivs, ivd,
          rt0, rt1, rq0, rq1, sgt0, sgt1, sgq0, sgq1, swt0, swt1, swq0, swq1):
        w = lax.axis_index("s") * 2 + lax.axis_index("c")
        base = w * per_w
        pltpu.sync_copy(s_hbm.at[pl.ds(start + base, per_w)], ivs)
        pltpu.sync_copy(d_hbm.at[pl.ds(start + base, per_w)], ivd)

        def gstart(i, rt, rq, st, sq):
            pltpu.async_copy(h_hbm.at[ivs.at[pl.ds(i * _KG, _KG)]], rt, st)
            pltpu.async_copy(h_hbm.at[ivd.at[pl.ds(i * _KG, _KG)]], rq, sq)

        def gwait(i, rt, rq, st, sq):
            pltpu.make_async_copy(h_hbm.at[ivs.at[pl.ds(i * _KG, _KG)]], rt, st).wait()
            pltpu.make_async_copy(h_hbm.at[ivd.at[pl.ds(i * _KG, _KG)]], rq, sq).wait()

        def wstart(i, rt, rq, st, sq):
            pltpu.async_copy(rt, hs_hbm.at[pl.ds(base + i * _KG, _KG)], st)
            pltpu.async_copy(rq, hd_hbm.at[pl.ds(base + i * _KG, _KG)], sq)

        def wwait(i, rt, rq, st, sq):
            pltpu.make_async_copy(rt, hs_hbm.at[pl.ds(base + i * _KG, _KG)], st).wait()
            pltpu.make_async_copy(rq, hd_hbm.at[pl.ds(base + i * _KG, _KG)], sq).wait()

        gstart(0, rt0, rq0, sgt0, sgq0)
        gstart(1, rt1, rq1, sgt1, sgq1)

        @pl.loop(0, nchg // 2 - 1)
        def _(p):
            i0 = 2 * p
            gwait(i0, rt0, rq0, sgt0, sgq0)
            wstart(i0, rt0, rq0, swt0, swq0)
            gwait(i0 + 1, rt1, rq1, sgt1, sgq1)
            wstart(i0 + 1, rt1, rq1, swt1, swq1)
            wwait(i0, rt0, rq0, swt0, swq0)
            gstart(i0 + 2, rt0, rq0, sgt0, sgq0)
            wwait(i0 + 1, rt1, rq1, swt1, swq1)
            gstart(i0 + 3, rt1, rq1, sgt1, sgq1)

        i0 = nchg - 2
        gwait(i0, rt0, rq0, sgt0, sgq0)
        wstart(i0, rt0, rq0, swt0, swq0)
        gwait(i0 + 1, rt1, rq1, sgt1, sgq1)
        wstart(i0 + 1, rt1, rq1, swt1, swq1)
        wwait(i0, rt0, rq0, swt0, swq0)
        wwait(i0 + 1, rt1, rq1, swt1, swq1)

    return k(H, srcp, dstp)


def _sc_scatter_add(msg2, dst3, init2):
    ep = msg2.shape[1]
    per_s = ep // 16
    ch_s = per_s // _KCH

    @functools.partial(
        pl.kernel, mesh=_sc_mesh(),
        out_type=jax.ShapeDtypeStruct((2, _NPAD, 128), _F32),
        scratch_types=[pltpu.VMEM((ch_s, _KCH), jnp.int32),
                       pltpu.VMEM((_KCH, 128), _F32),
                       pltpu.VMEM((_KCH, 128), _F32),
                       pltpu.VMEM_SHARED((_NPAD, 128), _F32),
                       pltpu.SemaphoreType.DMA, pltpu.SemaphoreType.DMA],
    )
    def k(m_hbm, d3_hbm, z_hbm, agg_hbm, iv2d, rv0, rv1, acc, sm0, sm1):
        c = lax.axis_index("c")
        s = lax.axis_index("s")

        @pl.when(s == 0)
        def _():
            pltpu.sync_copy(z_hbm.at[c], acc)

        pltpu.sync_copy(d3_hbm.at[s], iv2d)
        plsc.subcore_barrier()
        mbase = s * per_s

        def mstart(i, rv, sm):
            pltpu.async_copy(m_hbm.at[c].at[pl.ds(mbase + i * _KCH, _KCH)], rv, sm)

        def mwait(i, rv, sm):
            pltpu.make_async_copy(
                m_hbm.at[c].at[pl.ds(mbase + i * _KCH, _KCH)], rv, sm).wait()

        def addi(i, rv):
            pltpu.sync_copy(rv, acc.at[iv2d.at[i]], add=True)

        mstart(0, rv0, sm0)
        mstart(1, rv1, sm1)

        @pl.loop(0, ch_s // 2 - 1)
        def _(p):
            i0 = 2 * p
            mwait(i0, rv0, sm0)
            addi(i0, rv0)
            mstart(i0 + 2, rv0, sm0)
            mwait(i0 + 1, rv1, sm1)
            addi(i0 + 1, rv1)
            mstart(i0 + 3, rv1, sm1)

        i0 = ch_s - 2
        mwait(i0, rv0, sm0)
        addi(i0, rv0)
        mwait(i0 + 1, rv1, sm1)
        addi(i0 + 1, rv1)

        plsc.subcore_barrier()
        rows = _NPAD // 16
        pltpu.sync_copy(acc.at[pl.ds(s * rows, rows)],
                        agg_hbm.at[c].at[pl.ds(s * rows, rows)])

    return k(msg2, dst3, init2)


def _dot(a, b):
    return jnp.dot(a, b, preferred_element_type=_F32)


def _embed_body(xs_ref, ea_ref, we_ref, be_ref, wn_ref, bn_ref, eo_ref, msg_ref):
    t = pl.program_id(0)
    e = _dot(ea_ref[...], we_ref[...]) + be_ref[...]
    eo_ref[...] = e
    hsrc = xs_ref[...] * wn_ref[...] + bn_ref[...]
    msg = jnp.maximum(hsrc + e, 0.0)
    ids = lax.broadcasted_iota(jnp.int32, (_TE, 1), 0) + t * _TE
    msg = jnp.where(ids < _E, msg, 0.0)
    msg_ref[0] = msg[:, :128]
    msg_ref[1] = msg[:, 128:]


def _edge_mlp_body(hs_ref, hd_ref, e_ref, a_ref, b_ref, c_ref, w2_ref,
                   be1_ref, be2_ref, msg_ref, *, start):
    t = pl.program_id(0)
    hs = hs_ref[...]
    e = e_ref[...]
    tt = jnp.maximum(
        _dot(hs, a_ref[...]) + _dot(hd_ref[...], b_ref[...])
        + _dot(e, c_ref[...]) + be1_ref[...], 0.0)
    eu = _dot(tt, w2_ref[...]) + be2_ref[...]
    e1 = (e + eu) * 0.5
    msg = jnp.maximum(hs + e1, 0.0)
    ids = lax.broadcasted_iota(jnp.int32, (_TEM, 1), 0) + start + t * _TEM
    msg = jnp.where(ids < _E, msg, 0.0)
    msg_ref[0] = msg[:, :128]
    msg_ref[1] = msg[:, 128:]


def _z_from_x_body(x_ref, agg_ref, wn_ref, bn_ref, w1_ref, b1_ref, w2_ref,
                   b2_ref, z_ref, st_ref):
    h = x_ref[...] * wn_ref[...] + bn_ref[...]
    _z_common(h, agg_ref, w1_ref, b1_ref, w2_ref, b2_ref, z_ref, st_ref)


def _z_from_h_body(h_ref, agg_ref, w1_ref, b1_ref, w2_ref, b2_ref, z_ref, st_ref):
    _z_common(h_ref[...], agg_ref, w1_ref, b1_ref, w2_ref, b2_ref, z_ref, st_ref)


def _z_common(h, agg_ref, w1_ref, b1_ref, w2_ref, b2_ref, z_ref, st_ref):
    agg = jnp.concatenate([agg_ref[0], agg_ref[1]], axis=1)
    zin = h + agg
    z1 = jnp.maximum(_dot(zin, w1_ref[...]) + b1_ref[...], 0.0)
    z = _dot(z1, w2_ref[...]) + b2_ref[...]
    z_ref[...] = z
    t = pl.program_id(0)

    @pl.when(t == 0)
    def _():
        st_ref[...] = jnp.zeros_like(st_ref)

    st_ref[0:1, :] += jnp.sum(z, axis=0, keepdims=True)
    st_ref[1:2, :] += jnp.sum(z * z, axis=0, keepdims=True)


def _bn_h(z_ref, st_ref, hprev):
    mu = st_ref[0:1, :] / _N
    var = st_ref[1:2, :] / _N - mu * mu
    zn = (z_ref[...] - mu) / jnp.sqrt(var + 1e-5)
    return (hprev + jnp.maximum(zn, 0.0)) * 0.5


def _ht_body(x_ref, z_ref, st_ref, wn_ref, bn_ref, h_ref):
    h = x_ref[...] * wn_ref[...] + bn_ref[...]
    h_ref[...] = _bn_h(z_ref, st_ref, h)


def _head_body(h1_ref, z_ref, st_ref, wc1_ref, bc1_ref, wc2_ref, bc2_ref, o_ref):
    h2 = _bn_h(z_ref, st_ref, h1_ref[...])
    y = jnp.maximum(_dot(h2, wc1_ref[...]) + bc1_ref[...], 0.0)
    o_ref[...] = _dot(y, wc2_ref[...]) + bc2_ref[...]


def _const(shape):
    return pl.BlockSpec(shape, lambda t: tuple(0 for _ in shape))


def _tc_embed(xs, eap, W_edge, be, wn, bn):
    nt = _EPAD // _TE
    return pl.pallas_call(
        _embed_body,
        grid=(nt,),
        in_specs=[pl.BlockSpec((_TE, 1), lambda t: (t, 0)),
                  pl.BlockSpec((_TE, 16), lambda t: (t, 0)),
                  _const((16, 256)), _const((1, 256)),
                  _const((1, 256)), _const((1, 256))],
        out_specs=[pl.BlockSpec((_TE, 256), lambda t: (t, 0)),
                   pl.BlockSpec((2, _TE, 128), lambda t: (0, t, 0))],
        out_shape=[jax.ShapeDtypeStruct((_EPAD, 256), _F32),
                   jax.ShapeDtypeStruct((2, _EPAD, 128), _F32)],
    )(xs, eap, W_edge, be, wn, bn)


def _tc_edge_mlp(Hs, Hd, e0, A, B, C, We2i, be1i, be2i, start):
    count = Hs.shape[0]
    nt = count // _TEM
    toff = start // _TEM
    return pl.pallas_call(
        functools.partial(_edge_mlp_body, start=start),
        grid=(nt,),
        in_specs=[pl.BlockSpec((_TEM, 256), lambda t: (t, 0)),
                  pl.BlockSpec((_TEM, 256), lambda t: (t, 0)),
                  pl.BlockSpec((_TEM, 256), lambda t: (t + toff, 0)),
                  _const((256, 256)), _const((256, 256)), _const((256, 256)),
                  _const((256, 256)),
                  _const((1, 256)), _const((1, 256))],
        out_specs=[pl.BlockSpec((2, _TEM, 128), lambda t: (0, t, 0))],
        out_shape=[jax.ShapeDtypeStruct((2, count, 128), _F32)],
    )(Hs, Hd, e0, A, B, C, We2i, be1i, be2i)[0]


def _tc_z(h_or_x, agg, wn_bn, w1, b1r, w2, b2r, from_x):
    if from_x:
        body = _z_from_x_body
        first_specs = [pl.BlockSpec((_TN, 1), lambda t: (t, 0))]
        wspecs = [_const((1, 256)), _const((1, 256))]
        wargs = list(wn_bn)
    else:
        body = _z_from_h_body
        first_specs = [pl.BlockSpec((_TN, 256), lambda t: (t, 0))]
        wspecs = []
        wargs = []
    return pl.pallas_call(
        body,
        grid=(_NT,),
        in_specs=first_specs
        + [pl.BlockSpec((2, _TN, 128), lambda t: (0, t, 0))]
        + wspecs
        + [_const((256, 256)), _const((1, 256)),
           _const((256, 256)), _const((1, 256))],
        out_specs=[pl.BlockSpec((_TN, 256), lambda t: (t, 0)),
                   _const((8, 256))],
        out_shape=[jax.ShapeDtypeStruct((_N, 256), _F32),
                   jax.ShapeDtypeStruct((8, 256), _F32)],
    )(h_or_x, agg, *wargs, w1, b1r, w2, b2r)


def _tc_ht(x, z0, st0, wn, bn):
    return pl.pallas_call(
        _ht_body,
        grid=(_NT,),
        in_specs=[pl.BlockSpec((_TN, 1), lambda t: (t, 0)),
                  pl.BlockSpec((_TN, 256), lambda t: (t, 0)),
                  _const((8, 256)),
                  _const((1, 256)), _const((1, 256))],
        out_specs=[pl.BlockSpec((_TN, 256), lambda t: (t, 0))],
        out_shape=[jax.ShapeDtypeStruct((_N, 256), _F32)],
    )(x, z0, st0, wn, bn)[0]


def _tc_head(H1, z1, st1, Wc1, bc1r, Wc2p, bc2r):
    return pl.pallas_call(
        _head_body,
        grid=(_NT,),
        in_specs=[pl.BlockSpec((_TN, 256), lambda t: (t, 0)),
                  pl.BlockSpec((_TN, 256), lambda t: (t, 0)),
                  _const((8, 256)),
                  _const((256, 256)), _const((1, 256)),
                  _const((256, 128)), _const((1, 128))],
        out_specs=[pl.BlockSpec((_TN, 128), lambda t: (t, 0))],
        out_shape=[jax.ShapeDtypeStruct((_N, 128), _F32)],
    )(H1, z1, st1, Wc1, bc1r, Wc2p, bc2r)[0]


def kernel(x, edge_index, edge_attr, W_node, b_node, W_edge, b_edge, W1, b1,
           W2, b2, We1, be1, We2, be2, Wc1, bc1, Wc2, bc2):
    pad = _EPAD - _E
    padidx = jnp.arange(pad, dtype=jnp.int32) % _N
    srcp = jnp.concatenate([edge_index[0].astype(jnp.int32), padidx])
    dstp = jnp.concatenate([edge_index[1].astype(jnp.int32), padidx])
    eap = jnp.pad(edge_attr, ((0, pad), (0, 0)))
    eh = _EPAD // 2
    dst3 = dstp.reshape(16, _CH_S, _KCH)
    dst3a = dstp[:eh].reshape(16, _CH_S // 2, _KCH)
    dst3b = dstp[eh:].reshape(16, _CH_S // 2, _KCH)
    zrows2 = jnp.zeros((2, _NPAD, 128), _F32)

    wn = W_node
    bn = b_node[None, :]
    ber = b_edge[None, :]
    A0, B0, C0 = We1[0, 0:256], We1[0, 256:512], We1[0, 512:768]
    Wc2p = jnp.pad(Wc2, ((0, 0), (0, 128 - Wc2.shape[1])))
    bc2r = jnp.pad(bc2, (0, 128 - bc2.shape[0]))[None, :]

    xs = _sc_gather_x(x[:, 0], srcp)
    e0, msg0 = _tc_embed(xs[:, None], eap, W_edge, ber, wn, bn)
    agg0 = _sc_scatter_add(msg0, dst3, zrows2)
    z0, st0 = _tc_z(x, agg0, (wn, bn), W1[0], b1[0][None], W2[0], b2[0][None],
                    from_x=True)
    H1 = _tc_ht(x, z0, st0, wn, bn)

    HsA, HdA = _sc_gather_hh(H1, srcp, dstp, 0, eh)
    HsB, HdB = _sc_gather_hh(H1, srcp, dstp, eh, eh)
    msg1a = _tc_edge_mlp(HsA, HdA, e0, A0, B0, C0, We2[0],
                         be1[0][None], be2[0][None], 0)
    msg1b = _tc_edge_mlp(HsB, HdB, e0, A0, B0, C0, We2[0],
                         be1[0][None], be2[0][None], eh)
    agg1p = _sc_scatter_add(msg1a, dst3a, zrows2)
    agg1 = _sc_scatter_add(msg1b, dst3b, agg1p)
    z1, st1 = _tc_z(H1, agg1, None, W1[1], b1[1][None], W2[1], b2[1][None],
                    from_x=False)
    outp = _tc_head(H1, z1, st1, Wc1, bc1[None], Wc2p, bc2r)
    return outp[:, :Wc2.shape[1]]
